# Initial kernel scaffold; baseline (speedup 1.0000x reference)
import jax
import jax.numpy as jnp
from jax.experimental import pallas as pl


def kernel(edge_rep, cycle_rep, src, dst, W1, g1, b1, W2a, g2a, b2a, W2b, g2b, b2b, Wla, gla, bla, Wlb, glb, blb, eps1, eps2):
    raise NotImplementedError("write your pallas kernel here")


# speedup vs baseline: 1.0242x; 1.0242x over previous
import functools

import jax
import jax.numpy as jnp
from jax.experimental import pallas as pl
from jax.experimental.pallas import tpu as pltpu

H = 128
BN = 1000


def _mm_body(A_ref, B_ref, a_ref, b_ref, W_ref, Y_ref, s_ref, q_ref, *,
             affine, relu, use_B, stats):
    U = A_ref[...]
    if affine:
        U = U * a_ref[...] + b_ref[...]
    if relu:
        U = jnp.maximum(U, 0.0)
    if use_B:
        U = U + B_ref[...]
    Y = jnp.dot(U, W_ref[...], preferred_element_type=jnp.float32)
    Y_ref[...] = Y
    if stats:
        s_ref[...] = jnp.sum(Y, axis=0, keepdims=True)[None]
        q_ref[...] = jnp.sum(Y * Y, axis=0, keepdims=True)[None]


def _mm(A, W, B=None, a=None, b=None, relu=False, stats=False):
    N = A.shape[0]
    nb = N // BN
    use_B = B is not None
    affine = a is not None
    if B is None:
        B = jnp.zeros((1, H), jnp.float32)
    if a is None:
        a = jnp.zeros((1, H), jnp.float32)
        b = jnp.zeros((1, H), jnp.float32)
    a = a.reshape(1, H)
    b = b.reshape(1, H)
    body = functools.partial(_mm_body, affine=affine, relu=relu, use_B=use_B,
                             stats=stats)
    grid = (nb,)
    in_specs = [
        pl.BlockSpec((BN, H), lambda i: (i, 0)),
        pl.BlockSpec((BN, H), lambda i: (i, 0)) if use_B
        else pl.BlockSpec((1, H), lambda i: (0, 0)),
        pl.BlockSpec((1, H), lambda i: (0, 0)),
        pl.BlockSpec((1, H), lambda i: (0, 0)),
        pl.BlockSpec((H, H), lambda i: (0, 0)),
    ]
    out_shape = [
        jax.ShapeDtypeStruct((N, H), jnp.float32),
        jax.ShapeDtypeStruct((nb, 1, H), jnp.float32),
        jax.ShapeDtypeStruct((nb, 1, H), jnp.float32),
    ]
    out_specs = [
        pl.BlockSpec((BN, H), lambda i: (i, 0)),
        pl.BlockSpec((1, 1, H), lambda i: (i, 0, 0)),
        pl.BlockSpec((1, 1, H), lambda i: (i, 0, 0)),
    ]
    Y, s, q = pl.pallas_call(
        body, grid=grid, in_specs=in_specs, out_specs=out_specs,
        out_shape=out_shape)(A, B, a, b, W)
    if not stats:
        return Y, None, None
    return Y, jnp.sum(s, axis=(0, 1)), jnp.sum(q, axis=(0, 1))


def _ew_body(A_ref, a_ref, b_ref, Y_ref):
    Y_ref[...] = jnp.maximum(A_ref[...] * a_ref[...] + b_ref[...], 0.0)


def _ew_relu(A, a, b):
    N = A.shape[0]
    return pl.pallas_call(
        _ew_body, grid=(N // BN,),
        in_specs=[pl.BlockSpec((BN, H), lambda i: (i, 0)),
                  pl.BlockSpec((1, H), lambda i: (0, 0)),
                  pl.BlockSpec((1, H), lambda i: (0, 0))],
        out_specs=pl.BlockSpec((BN, H), lambda i: (i, 0)),
        out_shape=jax.ShapeDtypeStruct((N, H), jnp.float32),
    )(A, a.reshape(1, H), b.reshape(1, H))


def _bn_coeffs(s, q, n, g, b):
    mean = s / n
    var = q / n - mean * mean
    a = g / jnp.sqrt(var + 1e-5)
    return a, b - mean * a


def kernel(edge_rep, cycle_rep, src, dst, W1, g1, b1, W2a, g2a, b2a,
           W2b, g2b, b2b, Wla, gla, bla, Wlb, glb, blb, eps1, eps2):
    E, C, P = edge_rep.shape[0], cycle_rep.shape[0], src.shape[0]

    Xe, _, _ = _mm(edge_rep, W1[:H])
    Yc, _, _ = _mm(cycle_rep, W1[H:])

    z = Xe[src] + Yc[dst]
    mean = jnp.mean(z, axis=0)
    var = jnp.mean(z * z, axis=0) - mean * mean
    a1 = g1 / jnp.sqrt(var + 1e-5)
    bb1 = b1 - mean * a1
    msg = jax.nn.relu(z * a1 + bb1)

    S1 = jax.ops.segment_sum(edge_rep[src], dst, num_segments=C)
    S2 = jax.ops.segment_sum(msg, dst, num_segments=C)
    lvl = jax.ops.segment_sum(S2[dst] - msg, src, num_segments=E)

    one = jnp.ones((1, H), jnp.float32)
    zero = jnp.zeros((1, H), jnp.float32)

    M1, s, q = _mm(edge_rep, W2a, B=lvl, a=(1.0 + eps1) * one, b=zero,
                   stats=True)
    a, b = _bn_coeffs(s, q, E, g2a, b2a)
    M2, s, q = _mm(M1, W2b, a=a, b=b, relu=True, stats=True)
    a, b = _bn_coeffs(s, q, E, g2b, b2b)
    e_out = _ew_relu(M2, a, b)

    N1, s, q = _mm(cycle_rep, Wla, B=S1, a=(1.0 + eps2) * one, b=zero,
                   stats=True)
    a, b = _bn_coeffs(s, q, C, gla, bla)
    N2, s, q = _mm(N1, Wlb, a=a, b=b, relu=True, stats=True)
    a, b = _bn_coeffs(s, q, C, glb, blb)
    c_out = _ew_relu(N2, a, b)

    return (e_out, c_out)


# speedup vs baseline: 2.4636x; 2.4054x over previous
import functools

import jax
import jax.numpy as jnp
from jax import lax
from jax.experimental import pallas as pl
from jax.experimental.pallas import tpu as pltpu
from jax.experimental.pallas import tpu_sc as plsc

H = 128
BN = 1000

N_E = 100000
N_C = 30000
N_P = 600000
NW = 32
CH1 = 480
NCH1 = N_P // CH1


def _sc_mesh():
    return plsc.VectorSubcoreMesh(core_axis_name="c", subcore_axis_name="s")


def _pairs_k1(Xe, Yc, src, dst):
    @functools.partial(
        pl.kernel,
        out_type=[jax.ShapeDtypeStruct((N_P, H), jnp.float32),
                  jax.ShapeDtypeStruct((NW, 2, H), jnp.float32)],
        mesh=_sc_mesh(),
        compiler_params=pltpu.CompilerParams(needs_layout_passes=False),
        scratch_types=[
            pltpu.VMEM((CH1,), jnp.int32),
            pltpu.VMEM((CH1,), jnp.int32),
            pltpu.VMEM((CH1, H), jnp.float32),
            pltpu.VMEM((CH1, H), jnp.float32),
            pltpu.VMEM((2, H), jnp.float32),
            pltpu.SemaphoreType.DMA,
            pltpu.SemaphoreType.DMA,
        ],
    )
    def k1(xe_hbm, yc_hbm, src_hbm, dst_hbm, z_hbm, st_hbm,
           si_v, di_v, xe_v, yc_v, st_v, sem1, sem2):
        cid = lax.axis_index("c")
        sid = lax.axis_index("s")
        wid = sid * 2 + cid
        zero = jnp.zeros((16,), jnp.float32)
        carry0 = (zero,) * 16

        def chunk_body(t, carry):
            k = wid + t * NW
            base = k * CH1
            pltpu.sync_copy(src_hbm.at[pl.ds(base, CH1)], si_v)
            pltpu.sync_copy(dst_hbm.at[pl.ds(base, CH1)], di_v)
            c1 = pltpu.async_copy(xe_hbm.at[si_v], xe_v, sem1)
            c2 = pltpu.async_copy(yc_hbm.at[di_v], yc_v, sem2)
            c1.wait()
            c2.wait()

            def row_body(r, rc):
                zs = []
                for c in range(8):
                    x = xe_v[r, pl.ds(c * 16, 16)]
                    y = yc_v[r, pl.ds(c * 16, 16)]
                    z = x + y
                    xe_v[r, pl.ds(c * 16, 16)] = z
                    zs.append(z)
                out = []
                for c in range(8):
                    out.append(rc[c] + zs[c])
                for c in range(8):
                    out.append(rc[8 + c] + zs[c] * zs[c])
                return tuple(out)

            carry = lax.fori_loop(0, CH1, row_body, carry)
            pltpu.sync_copy(xe_v, z_hbm.at[pl.ds(base, CH1)])
            return carry

        ntr = (NCH1 - wid + NW - 1) // NW
        carry = lax.fori_loop(0, ntr, chunk_body, carry0)
        for c in range(8):
            st_v[0, pl.ds(c * 16, 16)] = carry[c]
            st_v[1, pl.ds(c * 16, 16)] = carry[8 + c]
        pltpu.sync_copy(st_v, st_hbm.at[wid])

    return k1(Xe, Yc, src, dst)


CH2 = 480
NCH2 = N_P // CH2
B = 96
STG = 592
CQ = 5008
AQ = 5120
ZQ = 320
ER = 12504
AR = 12544
ZR = 392


def _iota16():
    return lax.iota(jnp.int32, 16)


def _memset_zero(buf, rows):
    zero = jnp.zeros((16,), jnp.float32)

    def body(r, _):
        for c in range(8):
            buf[r, pl.ds(c * 16, 16)] = zero
        return 0

    lax.fori_loop(0, rows, body, 0)


def _seg_dst_k2(z, src, dst, edge_rep, ab):
    @functools.partial(
        pl.kernel,
        out_type=[jax.ShapeDtypeStruct((N_C, H), jnp.float32),
                  jax.ShapeDtypeStruct((N_C, H), jnp.float32)],
        mesh=_sc_mesh(),
        compiler_params=pltpu.CompilerParams(needs_layout_passes=False),
        scratch_types=[
            pltpu.VMEM((CH2,), jnp.int32),
            pltpu.VMEM((CH2,), jnp.int32),
            pltpu.VMEM((2, H), jnp.float32),
            pltpu.VMEM((STG,), jnp.int32),
            pltpu.VMEM((STG,), jnp.int32),
            pltpu.VMEM((STG,), jnp.int32),
            pltpu.VMEM((1, B), jnp.int32),
            pltpu.VMEM((B, H), jnp.float32),
            pltpu.VMEM((B, H), jnp.float32),
            pltpu.VMEM_SHARED((AQ, H), jnp.float32),
            pltpu.VMEM_SHARED((AQ, H), jnp.float32),
            pltpu.SemaphoreType.DMA,
            pltpu.SemaphoreType.DMA,
        ],
    )
    def k2(z_hbm, src_hbm, dst_hbm, er_hbm, ab_hbm, s1_hbm, s2_hbm,
           si_v, di_v, ab_v, stg_p, stg_s, stg_d, idx2d, zb_v, eb_v,
           acc1, acc2, sem1, sem2):
        cid = lax.axis_index("c")
        sid = lax.axis_index("s")
        iota = _iota16()
        pltpu.sync_copy(ab_hbm, ab_v)

        for phase in range(3):
            q = phase * 2 + cid
            lo = q * CQ
            _memset_zero(zb_v, B)
            z0 = sid * ZQ
            for acc in (acc1, acc2):
                for i in range(3):
                    pltpu.sync_copy(zb_v, acc.at[pl.ds(z0 + i * B, B)])
                pltpu.sync_copy(zb_v.at[pl.ds(0, 32)], acc.at[pl.ds(z0 + 288, 32)])
            plsc.subcore_barrier()

            def fire(j, _):
                off = j * B
                for kk in range(B // 16):
                    idx2d[0, pl.ds(kk * 16, 16)] = stg_d[pl.ds(off + kk * 16, 16)]
                cz = pltpu.async_copy(z_hbm.at[stg_p.at[pl.ds(off, B)]], zb_v, sem1)
                ce = pltpu.async_copy(er_hbm.at[stg_s.at[pl.ds(off, B)]], eb_v, sem2)
                cz.wait()
                ce.wait()

                def mrow(r, _):
                    for c in range(8):
                        zz = zb_v[r, pl.ds(c * 16, 16)]
                        aa = ab_v[0, pl.ds(c * 16, 16)]
                        bb = ab_v[1, pl.ds(c * 16, 16)]
                        zb_v[r, pl.ds(c * 16, 16)] = jnp.maximum(zz * aa + bb, 0.0)
                    return 0

                lax.fori_loop(0, B, mrow, 0)
                pltpu.sync_copy(zb_v, acc2.at[idx2d.at[0]], add=True)
                pltpu.sync_copy(eb_v, acc1.at[idx2d.at[0]], add=True)
                return 0

            def chunk(t, cnt):
                k = sid + t * 16
                base = k * CH2
                pltpu.sync_copy(src_hbm.at[pl.ds(base, CH2)], si_v)
                pltpu.sync_copy(dst_hbm.at[pl.ds(base, CH2)], di_v)

                def vloop(v, cn):
                    s = si_v[pl.ds(v * 16, 16)]
                    d = di_v[pl.ds(v * 16, 16)]
                    dl = d - lo
                    m = (dl >= 0) & (dl < CQ)
                    mi = m.astype(jnp.int32)
                    rank = plsc.cumsum(mi) - mi
                    pos = jnp.where(m, cn + rank, 576 + iota)
                    pidx = base + v * 16 + iota
                    plsc.store_scatter(stg_p, [pos], pidx)
                    plsc.store_scatter(stg_s, [pos], s)
                    plsc.store_scatter(stg_d, [pos], dl)
                    return cn + jnp.sum(mi)

                cnt = lax.fori_loop(0, CH2 // 16, vloop, cnt)
                nb = cnt // B
                lax.fori_loop(0, nb, fire, 0)
                roff = nb * B
                for kk in range(B // 16):
                    vp = stg_p[pl.ds(roff + kk * 16, 16)]
                    vs = stg_s[pl.ds(roff + kk * 16, 16)]
                    vd = stg_d[pl.ds(roff + kk * 16, 16)]
                    stg_p[pl.ds(kk * 16, 16)] = vp
                    stg_s[pl.ds(kk * 16, 16)] = vs
                    stg_d[pl.ds(kk * 16, 16)] = vd
                return cnt - nb * B

            cnt = lax.fori_loop(0, (NCH2 - sid + 15) // 16, chunk, 0)
            for kk in range(7):
                off = cnt + kk * 16
                stg_p[pl.ds(off, 16)] = iota
                stg_s[pl.ds(off, 16)] = iota
                stg_d[pl.ds(off, 16)] = CQ + iota
            fire(0, 0)
            plsc.subcore_barrier()
            w0 = sid * 312

            @pl.when((q < 5) | (sid < 15))
            def _():
                pltpu.sync_copy(acc1.at[pl.ds(w0, 312)], s1_hbm.at[pl.ds(lo + w0, 312)])
                pltpu.sync_copy(acc2.at[pl.ds(w0, 312)], s2_hbm.at[pl.ds(lo + w0, 312)])

            @pl.when((sid == 15) & (q < 5))
            def _():
                pltpu.sync_copy(acc1.at[pl.ds(4992, 16)], s1_hbm.at[pl.ds(lo + 4992, 16)])
                pltpu.sync_copy(acc2.at[pl.ds(4992, 16)], s2_hbm.at[pl.ds(lo + 4992, 16)])

            @pl.when((sid == 14) & (q == 5))
            def _():
                pltpu.sync_copy(acc1.at[pl.ds(4680, 280)], s1_hbm.at[pl.ds(lo + 4680, 280)])
                pltpu.sync_copy(acc2.at[pl.ds(4680, 280)], s2_hbm.at[pl.ds(lo + 4680, 280)])

            plsc.subcore_barrier()

    return k2(z, src, dst, edge_rep, ab)


def _seg_src_k3(z, src, dst, S2, ab):
    @functools.partial(
        pl.kernel,
        out_type=jax.ShapeDtypeStruct((N_E, H), jnp.float32),
        mesh=_sc_mesh(),
        compiler_params=pltpu.CompilerParams(needs_layout_passes=False),
        scratch_types=[
            pltpu.VMEM((CH2,), jnp.int32),
            pltpu.VMEM((CH2,), jnp.int32),
            pltpu.VMEM((2, H), jnp.float32),
            pltpu.VMEM((STG,), jnp.int32),
            pltpu.VMEM((STG,), jnp.int32),
            pltpu.VMEM((STG,), jnp.int32),
            pltpu.VMEM((1, B), jnp.int32),
            pltpu.VMEM((B, H), jnp.float32),
            pltpu.VMEM((B, H), jnp.float32),
            pltpu.VMEM_SHARED((AR, H), jnp.float32),
            pltpu.SemaphoreType.DMA,
            pltpu.SemaphoreType.DMA,
        ],
    )
    def k3(z_hbm, src_hbm, dst_hbm, s2_hbm, ab_hbm, lvl_hbm,
           si_v, di_v, ab_v, stg_p, stg_sl, stg_d, idx2d, zb_v, sb_v,
           acc, sem1, sem2):
        cid = lax.axis_index("c")
        sid = lax.axis_index("s")
        iota = _iota16()
        pltpu.sync_copy(ab_hbm, ab_v)

        for phase in range(4):
            r = phase * 2 + cid
            lo = r * ER

            @pl.when(r < 8)
            def _():
                _memset_zero(zb_v, B)
                z0 = sid * 784
                for i in range(8):
                    pltpu.sync_copy(zb_v, acc.at[pl.ds(z0 + i * B, B)])
                pltpu.sync_copy(zb_v.at[pl.ds(0, 16)], acc.at[pl.ds(z0 + 768, 16)])
                plsc.subcore_barrier()

                def fire(j, _):
                    off = j * B
                    for kk in range(B // 16):
                        idx2d[0, pl.ds(kk * 16, 16)] = stg_sl[pl.ds(off + kk * 16, 16)]
                    cz = pltpu.async_copy(z_hbm.at[stg_p.at[pl.ds(off, B)]], zb_v, sem1)
                    cs = pltpu.async_copy(s2_hbm.at[stg_d.at[pl.ds(off, B)]], sb_v, sem2)
                    cz.wait()
                    cs.wait()

                    def mrow(rr, _):
                        for c in range(8):
                            zz = zb_v[rr, pl.ds(c * 16, 16)]
                            aa = ab_v[0, pl.ds(c * 16, 16)]
                            bb = ab_v[1, pl.ds(c * 16, 16)]
                            ss = sb_v[rr, pl.ds(c * 16, 16)]
                            sb_v[rr, pl.ds(c * 16, 16)] = ss - jnp.maximum(zz * aa + bb, 0.0)
                        return 0

                    lax.fori_loop(0, B, mrow, 0)
                    pltpu.sync_copy(sb_v, acc.at[idx2d.at[0]], add=True)
                    return 0

                def chunk(t, cnt):
                    k = sid + t * 16
                    base = k * CH2
                    pltpu.sync_copy(src_hbm.at[pl.ds(base, CH2)], si_v)
                    pltpu.sync_copy(dst_hbm.at[pl.ds(base, CH2)], di_v)

                    def vloop(v, cn):
                        s = si_v[pl.ds(v * 16, 16)]
                        d = di_v[pl.ds(v * 16, 16)]
                        sl = s - lo
                        m = (sl >= 0) & (sl < ER)
                        mi = m.astype(jnp.int32)
                        rank = plsc.cumsum(mi) - mi
                        pos = jnp.where(m, cn + rank, 576 + iota)
                        pidx = base + v * 16 + iota
                        plsc.store_scatter(stg_p, [pos], pidx)
                        plsc.store_scatter(stg_sl, [pos], sl)
                        plsc.store_scatter(stg_d, [pos], d)
                        return cn + jnp.sum(mi)

                    cnt = lax.fori_loop(0, CH2 // 16, vloop, cnt)
                    nb = cnt // B
                    lax.fori_loop(0, nb, fire, 0)
                    roff = nb * B
                    for kk in range(B // 16):
                        vp = stg_p[pl.ds(roff + kk * 16, 16)]
                        vs = stg_sl[pl.ds(roff + kk * 16, 16)]
                        vd = stg_d[pl.ds(roff + kk * 16, 16)]
                        stg_p[pl.ds(kk * 16, 16)] = vp
                        stg_sl[pl.ds(kk * 16, 16)] = vs
                        stg_d[pl.ds(kk * 16, 16)] = vd
                    return cnt - nb * B

                cnt = lax.fori_loop(0, (NCH2 - sid + 15) // 16, chunk, 0)
                for kk in range(7):
                    off = cnt + kk * 16
                    stg_p[pl.ds(off, 16)] = iota
                    stg_sl[pl.ds(off, 16)] = ER + iota
                    stg_d[pl.ds(off, 16)] = iota
                fire(0, 0)
                plsc.subcore_barrier()

                w0 = sid * 776
                pltpu.sync_copy(acc.at[pl.ds(w0, 776)],
                                lvl_hbm.at[pl.ds(lo + w0, 776)])

                @pl.when((sid == 15) & (r < 7))
                def _w1b():
                    pltpu.sync_copy(acc.at[pl.ds(12416, 88)],
                                    lvl_hbm.at[pl.ds(lo + 12416, 88)])

                @pl.when((sid == 15) & (r == 7))
                def _w2b():
                    pltpu.sync_copy(acc.at[pl.ds(12416, 56)],
                                    lvl_hbm.at[pl.ds(lo + 12416, 56)])

                plsc.subcore_barrier()

    return k3(z, src, dst, S2, ab)


def _mm_body(A_ref, B_ref, a_ref, b_ref, W_ref, Y_ref, s_ref, q_ref, *,
             affine, relu, use_B, stats):
    U = A_ref[...]
    if affine:
        U = U * a_ref[...] + b_ref[...]
    if relu:
        U = jnp.maximum(U, 0.0)
    if use_B:
        U = U + B_ref[...]
    Y = jnp.dot(U, W_ref[...], preferred_element_type=jnp.float32)
    Y_ref[...] = Y
    if stats:
        s_ref[...] = jnp.sum(Y, axis=0, keepdims=True)[None]
        q_ref[...] = jnp.sum(Y * Y, axis=0, keepdims=True)[None]


def _mm(A, W, B=None, a=None, b=None, relu=False, stats=False):
    N = A.shape[0]
    nb = N // BN
    use_B = B is not None
    affine = a is not None
    if B is None:
        B = jnp.zeros((1, H), jnp.float32)
    if a is None:
        a = jnp.zeros((1, H), jnp.float32)
        b = jnp.zeros((1, H), jnp.float32)
    a = a.reshape(1, H)
    b = b.reshape(1, H)
    body = functools.partial(_mm_body, affine=affine, relu=relu, use_B=use_B,
                             stats=stats)
    grid = (nb,)
    in_specs = [
        pl.BlockSpec((BN, H), lambda i: (i, 0)),
        pl.BlockSpec((BN, H), lambda i: (i, 0)) if use_B
        else pl.BlockSpec((1, H), lambda i: (0, 0)),
        pl.BlockSpec((1, H), lambda i: (0, 0)),
        pl.BlockSpec((1, H), lambda i: (0, 0)),
        pl.BlockSpec((H, H), lambda i: (0, 0)),
    ]
    out_shape = [
        jax.ShapeDtypeStruct((N, H), jnp.float32),
        jax.ShapeDtypeStruct((nb, 1, H), jnp.float32),
        jax.ShapeDtypeStruct((nb, 1, H), jnp.float32),
    ]
    out_specs = [
        pl.BlockSpec((BN, H), lambda i: (i, 0)),
        pl.BlockSpec((1, 1, H), lambda i: (i, 0, 0)),
        pl.BlockSpec((1, 1, H), lambda i: (i, 0, 0)),
    ]
    Y, s, q = pl.pallas_call(
        body, grid=grid, in_specs=in_specs, out_specs=out_specs,
        out_shape=out_shape)(A, B, a, b, W)
    if not stats:
        return Y, None, None
    return Y, jnp.sum(s, axis=(0, 1)), jnp.sum(q, axis=(0, 1))


def _ew_body(A_ref, a_ref, b_ref, Y_ref):
    Y_ref[...] = jnp.maximum(A_ref[...] * a_ref[...] + b_ref[...], 0.0)


def _ew_relu(A, a, b):
    N = A.shape[0]
    return pl.pallas_call(
        _ew_body, grid=(N // BN,),
        in_specs=[pl.BlockSpec((BN, H), lambda i: (i, 0)),
                  pl.BlockSpec((1, H), lambda i: (0, 0)),
                  pl.BlockSpec((1, H), lambda i: (0, 0))],
        out_specs=pl.BlockSpec((BN, H), lambda i: (i, 0)),
        out_shape=jax.ShapeDtypeStruct((N, H), jnp.float32),
    )(A, a.reshape(1, H), b.reshape(1, H))


def _bn_coeffs(s, q, n, g, b):
    mean = s / n
    var = q / n - mean * mean
    a = g / jnp.sqrt(var + 1e-5)
    return a, b - mean * a


def kernel(edge_rep, cycle_rep, src, dst, W1, g1, b1, W2a, g2a, b2a,
           W2b, g2b, b2b, Wla, gla, bla, Wlb, glb, blb, eps1, eps2):
    E, C, P = edge_rep.shape[0], cycle_rep.shape[0], src.shape[0]

    Xe, _, _ = _mm(edge_rep, W1[:H])
    Yc, _, _ = _mm(cycle_rep, W1[H:])

    z, st = _pairs_k1(Xe, Yc, src, dst)
    sums = jnp.sum(st, axis=0)
    mean = sums[0] / N_P
    var = sums[1] / N_P - mean * mean
    a1 = g1 / jnp.sqrt(var + 1e-5)
    bb1 = b1 - mean * a1
    ab = jnp.stack([a1, bb1])

    S1, S2 = _seg_dst_k2(z, src, dst, edge_rep, ab)
    lvl = _seg_src_k3(z, src, dst, S2, ab)

    one = jnp.ones((1, H), jnp.float32)
    zero = jnp.zeros((1, H), jnp.float32)

    M1, s, q = _mm(edge_rep, W2a, B=lvl, a=(1.0 + eps1) * one, b=zero,
                   stats=True)
    a, b = _bn_coeffs(s, q, E, g2a, b2a)
    M2, s, q = _mm(M1, W2b, a=a, b=b, relu=True, stats=True)
    a, b = _bn_coeffs(s, q, E, g2b, b2b)
    e_out = _ew_relu(M2, a, b)

    N1, s, q = _mm(cycle_rep, Wla, B=S1, a=(1.0 + eps2) * one, b=zero,
                   stats=True)
    a, b = _bn_coeffs(s, q, C, gla, bla)
    N2, s, q = _mm(N1, Wlb, a=a, b=b, relu=True, stats=True)
    a, b = _bn_coeffs(s, q, C, glb, blb)
    c_out = _ew_relu(N2, a, b)

    return (e_out, c_out)
